# baseline (speedup 1.0000x reference)
import numpy as np
import jax
import jax.numpy as jnp
from jax import lax
from jax.experimental import pallas as pl
from jax.experimental.pallas import tpu as pltpu, tpu_sc as plsc

B = 16384
F = 22
D = 16
W = 100000
NC, NS, L = 2, 16, 16
HALF = B // NC
SLICE = HALF // NS
NROWS = 2200000
_STARTS = [(f * W) // 128 * 128 for f in range(F)]
_SHIFT = [f * W - _STARTS[f] for f in range(F)]
SLAB = ((W + 127) // 128 + 1) * 128
H0 = 50048
H1 = SLAB - H0
_LEN = [SLAB] * (F - 1) + [99968]
TAIL = 64
TAIL_START = NROWS - TAIL
_COLS = [0, 1, 4, 5, 12, 17, 18, 19, 20] + list(range(26, 39))


def _fm_body(xt_hbm, bias_hbm, tab_hbm, tail_hbm, out_hbm,
             slab_a, slab_b, idx_v, s_v, q_v, out_v, bias_v, tail_v, shared,
             sems, semi, semb):
    cid = lax.axis_index("c")
    sid = lax.axis_index("s")
    base = cid * HALF

    pltpu.sync_copy(bias_hbm, bias_v)
    pltpu.sync_copy(tail_hbm, tail_v)
    bias_vec = bias_v[pl.ds(0, L)]
    zero_i = jnp.zeros((L,), jnp.int32)
    zero16 = jnp.zeros((L,), jnp.float32)

    def zero_body(g, _):
        s_v[0, pl.ds(g * L, L)] = zero16
        q_v[0, pl.ds(g * L, L)] = zero16
        return 0
    lax.fori_loop(0, HALF // L, zero_body, 0)

    def issue_half(f, h, buf, sem):
        st = _STARTS[f] + (0 if h == 0 else H0)
        ln = (H0 if h == 0 else _LEN[f] - H0)
        return pltpu.async_copy(
            tab_hbm.at[pl.ds(sid, 1), pl.ds(st, ln)],
            buf.at[:, pl.ds(0, ln)], sem)

    def issue_idx(f):
        return pltpu.async_copy(
            xt_hbm.at[pl.ds(_COLS[f], 1), pl.ds(base, HALF)], idx_v, semi)

    def masked_pass(f, h, buf):
        lo = _SHIFT[f] - (0 if h == 0 else H0)
        cap = H0 if h == 0 else _LEN[f] - H0
        last = (f == F - 1) and (h == 1)

        def body(g, _):
            idx = idx_v[0, pl.ds(g * L, L)] + lo
            val = plsc.load_gather(
                slab_a if buf is slab_a else slab_b,
                [zero_i, jnp.clip(idx, 0, cap - 1)])
            ok = (idx >= 0) & (idx < cap)
            if last:
                trow = jnp.clip(idx - cap, 0, TAIL - 1)
                val_b = plsc.load_gather(tail_v,
                                         [zero_i, sid * TAIL + trow])
                val = jnp.where(ok, val, jnp.where(idx >= cap, val_b, 0.0))
            else:
                val = jnp.where(ok, val, 0.0)
            sl = pl.ds(g * L, L)
            plsc.addupdate(s_v.at[0, sl], val)
            plsc.addupdate(q_v.at[0, sl], val * val)
            return 0
        lax.fori_loop(0, HALF // L, body, 0)

    cpi = issue_idx(0)
    cpa = issue_half(0, 0, slab_a, sems)
    cpb = issue_half(0, 1, slab_b, semb)
    for f in range(F):
        cpi.wait()
        cpa.wait()
        masked_pass(f, 0, slab_a)
        if f + 1 < F:
            cpa = issue_half(f + 1, 0, slab_a, sems)
        cpb.wait()
        masked_pass(f, 1, slab_b)
        if f + 1 < F:
            cpb = issue_half(f + 1, 1, slab_b, semb)
            cpi = issue_idx(f + 1)

    def t_body(g, _):
        sl = pl.ds(g * L, L)
        s = s_v[0, sl]
        s_v[0, sl] = s + 0.5 * (s * s - q_v[0, sl])
        return 0
    lax.fori_loop(0, HALF // L, t_body, 0)

    def add_from_q(g, _):
        sl = pl.ds(g * L, L)
        plsc.addupdate(s_v.at[0, sl], q_v[0, sl])
        return 0

    for lo in (8, 4, 2, 1):
        @pl.when((sid >= lo) & (sid < 2 * lo))
        def _write():
            pltpu.sync_copy(s_v, shared.at[pl.ds(sid - lo, 1)])
        plsc.subcore_barrier()

        @pl.when(sid < lo)
        def _combine():
            pltpu.sync_copy(shared.at[pl.ds(sid, 1)], q_v)
            lax.fori_loop(0, HALF // L, add_from_q, 0)
        plsc.subcore_barrier()

    @pl.when(sid == 0)
    def _publish():
        pltpu.sync_copy(s_v, shared.at[pl.ds(0, 1)])
    plsc.subcore_barrier()
    pltpu.sync_copy(shared.at[pl.ds(0, 1), pl.ds(sid * SLICE, SLICE)],
                    slab_a.at[:, pl.ds(0, SLICE)])

    def out_body(g, _):
        z = slab_a[0, pl.ds(g * L, L)]
        y = 1.0 / (1.0 + jnp.exp(-(z + bias_vec)))
        out_v[pl.ds(g * L, L)] = y
        return 0
    lax.fori_loop(0, SLICE // L, out_body, 0)

    pltpu.sync_copy(out_v,
                    out_hbm.at[pl.ds(base + sid * SLICE, SLICE)])


@jax.jit
def _fm_call(xt, bias128, tabt, tail):
    mesh = plsc.VectorSubcoreMesh(core_axis_name="c", subcore_axis_name="s",
                                  num_cores=NC, num_subcores=NS)
    fn = pl.kernel(
        _fm_body,
        out_type=jax.ShapeDtypeStruct((B,), jnp.float32),
        mesh=mesh,
        compiler_params=pltpu.CompilerParams(needs_layout_passes=False,
                                             use_tc_tiling_on_sc=True),
        scratch_types=[
            pltpu.VMEM((1, H1), jnp.float32),
            pltpu.VMEM((1, H1), jnp.float32),
            pltpu.VMEM((1, HALF), jnp.int32),
            pltpu.VMEM((1, HALF), jnp.float32),
            pltpu.VMEM((1, HALF), jnp.float32),
            pltpu.VMEM((SLICE,), jnp.float32),
            pltpu.VMEM((128,), jnp.float32),
            pltpu.VMEM((1, TAIL * D), jnp.float32),
            pltpu.VMEM_SHARED((NS // 2, HALF), jnp.float32),
            pltpu.SemaphoreType.DMA,
            pltpu.SemaphoreType.DMA,
            pltpu.SemaphoreType.DMA,
        ],
    )
    return fn(xt, bias128, tabt, tail)


def kernel(x, additional, column, emb_table, bias):
    del additional, column
    xt = x.T
    tabt = emb_table.T
    bias128 = jnp.broadcast_to(bias.astype(jnp.float32), (128,))
    tail = emb_table[TAIL_START:, :].T.reshape(1, TAIL * D)
    return _fm_call(xt, bias128, tabt, tail)


# speedup vs baseline: 1.0677x; 1.0677x over previous
import numpy as np
import jax
import jax.numpy as jnp
from jax import lax
from jax.experimental import pallas as pl
from jax.experimental.pallas import tpu as pltpu, tpu_sc as plsc

B = 16384
F = 22
D = 16
W = 100000
NC, NS, L = 2, 16, 16
HALF = B // NC
SLICE = HALF // NS
NROWS = 2200000
_STARTS = [(f * W) // 128 * 128 for f in range(F)]
_SHIFT = [f * W - _STARTS[f] for f in range(F)]
SLAB = ((W + 127) // 128 + 1) * 128
_LEN = [SLAB] * (F - 1) + [99968]
TAIL = 64
TAIL_START = NROWS - TAIL
_COLS = [0, 1, 4, 5, 12, 17, 18, 19, 20] + list(range(26, 39))


def _fm_body(xt_hbm, bias_hbm, tab_hbm, tail_hbm, out_hbm,
             slab_v, idx_v, s_v, q_v, out_v, bias_v, tail_v, shared, sems,
             semi):
    cid = lax.axis_index("c")
    sid = lax.axis_index("s")
    base = cid * HALF

    pltpu.sync_copy(bias_hbm, bias_v)
    pltpu.sync_copy(tail_hbm, tail_v)
    bias_vec = bias_v[pl.ds(0, L)]
    zero_i = jnp.zeros((L,), jnp.int32)

    QT = 1
    QL = HALF

    NSTRIP = 4
    for f in range(F):
        ln = _LEN[f]
        sl_cuts = [ln * k // NSTRIP // 128 * 128 for k in range(NSTRIP)] + [ln]
        cps = []
        for k in range(NSTRIP):
            a, b = sl_cuts[k], sl_cuts[k + 1]
            cps.append(pltpu.async_copy(
                tab_hbm.at[pl.ds(sid, 1), pl.ds(_STARTS[f] + a, b - a)],
                slab_v.at[:, pl.ds(a, b - a)], sems))
        shift = _SHIFT[f]
        lim = _LEN[f]
        for qt in range(QT):
            cpi = pltpu.async_copy(
                xt_hbm.at[pl.ds(_COLS[f], 1),
                          pl.ds(base + qt * QL, QL)], idx_v, semi)
            if qt == 0:
                for cp in cps:
                    cp.wait()
            cpi.wait()
            qbase = qt * QL

            if f == 0:
                def init_body(g, _):
                    idx = idx_v[0, pl.ds(g * L, L)] + shift
                    val = plsc.load_gather(slab_v, [zero_i, idx])
                    sl = pl.ds(qbase + g * L, L)
                    s_v[0, sl] = val
                    q_v[0, sl] = val * val
                    return 0
                lax.fori_loop(0, QL // L, init_body, 0)
            elif f < F - 1:
                def acc_body(g, _):
                    idx = idx_v[0, pl.ds(g * L, L)] + shift
                    val = plsc.load_gather(slab_v, [zero_i, idx])
                    sl = pl.ds(qbase + g * L, L)
                    plsc.addupdate(s_v.at[0, sl], val)
                    plsc.addupdate(q_v.at[0, sl], val * val)
                    return 0
                lax.fori_loop(0, QL // L, acc_body, 0)
            else:
                def tail_body(g, _):
                    idx = idx_v[0, pl.ds(g * L, L)] + shift
                    in_slab = idx < lim
                    val_a = plsc.load_gather(
                        slab_v, [zero_i, jnp.minimum(idx, lim - 1)])
                    trow = jnp.clip(idx - lim, 0, TAIL - 1)
                    val_b = plsc.load_gather(tail_v,
                                             [zero_i, sid * TAIL + trow])
                    val = jnp.where(in_slab, val_a, val_b)
                    sl = pl.ds(qbase + g * L, L)
                    plsc.addupdate(s_v.at[0, sl], val)
                    plsc.addupdate(q_v.at[0, sl], val * val)
                    return 0
                lax.fori_loop(0, QL // L, tail_body, 0)

    def t_body(g, _):
        sl = pl.ds(g * L, L)
        s = s_v[0, sl]
        s_v[0, sl] = s + 0.5 * (s * s - q_v[0, sl])
        return 0
    lax.fori_loop(0, HALF // L, t_body, 0)

    def add_from_q(g, _):
        sl = pl.ds(g * L, L)
        plsc.addupdate(s_v.at[0, sl], q_v[0, sl])
        return 0

    for lo in (8, 4, 2, 1):
        @pl.when((sid >= lo) & (sid < 2 * lo))
        def _write():
            pltpu.sync_copy(s_v, shared.at[pl.ds(sid - lo, 1)])
        plsc.subcore_barrier()

        @pl.when(sid < lo)
        def _combine():
            pltpu.sync_copy(shared.at[pl.ds(sid, 1)], q_v)
            lax.fori_loop(0, HALF // L, add_from_q, 0)
        plsc.subcore_barrier()

    @pl.when(sid == 0)
    def _publish():
        pltpu.sync_copy(s_v, shared.at[pl.ds(0, 1)])
    plsc.subcore_barrier()
    pltpu.sync_copy(shared.at[pl.ds(0, 1), pl.ds(sid * SLICE, SLICE)],
                    slab_v.at[:, pl.ds(0, SLICE)])

    def out_body(g, _):
        z = slab_v[0, pl.ds(g * L, L)]
        y = 1.0 / (1.0 + jnp.exp(-(z + bias_vec)))
        out_v[pl.ds(g * L, L)] = y
        return 0
    lax.fori_loop(0, SLICE // L, out_body, 0)

    pltpu.sync_copy(out_v,
                    out_hbm.at[pl.ds(base + sid * SLICE, SLICE)])


@jax.jit
def _fm_call(xt, bias128, tabt, tail):
    mesh = plsc.VectorSubcoreMesh(core_axis_name="c", subcore_axis_name="s",
                                  num_cores=NC, num_subcores=NS)
    fn = pl.kernel(
        _fm_body,
        out_type=jax.ShapeDtypeStruct((B,), jnp.float32),
        mesh=mesh,
        compiler_params=pltpu.CompilerParams(needs_layout_passes=False,
                                             use_tc_tiling_on_sc=True),
        scratch_types=[
            pltpu.VMEM((1, SLAB), jnp.float32),
            pltpu.VMEM((1, HALF), jnp.int32),
            pltpu.VMEM((1, HALF), jnp.float32),
            pltpu.VMEM((1, HALF), jnp.float32),
            pltpu.VMEM((SLICE,), jnp.float32),
            pltpu.VMEM((128,), jnp.float32),
            pltpu.VMEM((1, TAIL * D), jnp.float32),
            pltpu.VMEM_SHARED((NS // 2, HALF), jnp.float32),
            pltpu.SemaphoreType.DMA,
            pltpu.SemaphoreType.DMA,
        ],
    )
    return fn(xt, bias128, tabt, tail)


def kernel(x, additional, column, emb_table, bias):
    del additional, column
    xt = x.T
    tabt = emb_table.T
    bias128 = jnp.broadcast_to(bias.astype(jnp.float32), (128,))
    tail = emb_table[TAIL_START:, :].T.reshape(1, TAIL * D)
    return _fm_call(xt, bias128, tabt, tail)


# speedup vs baseline: 1.0800x; 1.0116x over previous
import numpy as np
import jax
import jax.numpy as jnp
from jax import lax
from jax.experimental import pallas as pl
from jax.experimental.pallas import tpu as pltpu, tpu_sc as plsc

B = 16384
F = 22
D = 16
W = 100000
NC, NS, L = 2, 16, 16
HALF = B // NC
SLICE = HALF // NS
NROWS = 2200000
_STARTS = [(f * W) // 128 * 128 for f in range(F)]
_SHIFT = [f * W - _STARTS[f] for f in range(F)]
SLAB = ((W + 127) // 128 + 1) * 128
_LEN = [SLAB] * (F - 1) + [99968]
TAIL = 64
TAIL_START = NROWS - TAIL
_COLS = [0, 1, 4, 5, 12, 17, 18, 19, 20] + list(range(26, 39))


def _fm_body(xt_hbm, bias_hbm, tab_hbm, tail_hbm, out_hbm,
             slab_v, idx_v, s_v, q_v, out_v, bias_v, tail_v, shared, sems,
             semi):
    cid = lax.axis_index("c")
    sid = lax.axis_index("s")
    base = cid * HALF

    pltpu.sync_copy(bias_hbm, bias_v)
    pltpu.sync_copy(tail_hbm, tail_v)
    bias_vec = bias_v[pl.ds(0, L)]
    zero_i = jnp.zeros((L,), jnp.int32)

    QT = 1
    QL = HALF

    NSTRIP = 4
    for f in range(F):
        ln = _LEN[f]
        sl_cuts = [ln * k // NSTRIP // 128 * 128 for k in range(NSTRIP)] + [ln]
        cps = []
        for k in range(NSTRIP):
            a, b = sl_cuts[k], sl_cuts[k + 1]
            cps.append(pltpu.async_copy(
                tab_hbm.at[pl.ds(sid, 1), pl.ds(_STARTS[f] + a, b - a)],
                slab_v.at[:, pl.ds(a, b - a)], sems))
        shift = _SHIFT[f]
        lim = _LEN[f]
        for qt in range(QT):
            cpi = pltpu.async_copy(
                xt_hbm.at[pl.ds(_COLS[f], 1),
                          pl.ds(base + qt * QL, QL)], idx_v, semi)
            if qt == 0:
                for cp in cps:
                    cp.wait()
            cpi.wait()
            qbase = qt * QL

            if f == 0:
                def init_body(g, _):
                    idx = idx_v[0, pl.ds(g * L, L)] + shift
                    val = plsc.load_gather(slab_v, [zero_i, idx])
                    sl = pl.ds(qbase + g * L, L)
                    s_v[0, sl] = val
                    q_v[0, sl] = val * val
                    return 0
                lax.fori_loop(0, QL // L, init_body, 0)
            elif f < F - 1:
                def acc_body(g2, _):
                    for u in range(4):
                        g = g2 * 4 + u
                        idx = idx_v[0, pl.ds(g * L, L)] + shift
                        val = plsc.load_gather(slab_v, [zero_i, idx])
                        sl = pl.ds(qbase + g * L, L)
                        plsc.addupdate(s_v.at[0, sl], val)
                        plsc.addupdate(q_v.at[0, sl], val * val)
                    return 0
                lax.fori_loop(0, QL // L // 4, acc_body, 0)
            else:
                def tail_body(g, _):
                    idx = idx_v[0, pl.ds(g * L, L)] + shift
                    in_slab = idx < lim
                    val_a = plsc.load_gather(
                        slab_v, [zero_i, jnp.minimum(idx, lim - 1)])
                    trow = jnp.clip(idx - lim, 0, TAIL - 1)
                    val_b = plsc.load_gather(tail_v,
                                             [zero_i, sid * TAIL + trow])
                    val = jnp.where(in_slab, val_a, val_b)
                    sl = pl.ds(qbase + g * L, L)
                    plsc.addupdate(s_v.at[0, sl], val)
                    plsc.addupdate(q_v.at[0, sl], val * val)
                    return 0
                lax.fori_loop(0, QL // L, tail_body, 0)

    def t_body(g, _):
        sl = pl.ds(g * L, L)
        s = s_v[0, sl]
        s_v[0, sl] = s + 0.5 * (s * s - q_v[0, sl])
        return 0
    lax.fori_loop(0, HALF // L, t_body, 0)

    def add_from_q(g, _):
        sl = pl.ds(g * L, L)
        plsc.addupdate(s_v.at[0, sl], q_v[0, sl])
        return 0

    for lo in (8, 4, 2, 1):
        @pl.when((sid >= lo) & (sid < 2 * lo))
        def _write():
            pltpu.sync_copy(s_v, shared.at[pl.ds(sid - lo, 1)])
        plsc.subcore_barrier()

        @pl.when(sid < lo)
        def _combine():
            pltpu.sync_copy(shared.at[pl.ds(sid, 1)], q_v)
            lax.fori_loop(0, HALF // L, add_from_q, 0)
        plsc.subcore_barrier()

    @pl.when(sid == 0)
    def _publish():
        pltpu.sync_copy(s_v, shared.at[pl.ds(0, 1)])
    plsc.subcore_barrier()
    pltpu.sync_copy(shared.at[pl.ds(0, 1), pl.ds(sid * SLICE, SLICE)],
                    slab_v.at[:, pl.ds(0, SLICE)])

    def out_body(g, _):
        z = slab_v[0, pl.ds(g * L, L)]
        y = 1.0 / (1.0 + jnp.exp(-(z + bias_vec)))
        out_v[pl.ds(g * L, L)] = y
        return 0
    lax.fori_loop(0, SLICE // L, out_body, 0)

    pltpu.sync_copy(out_v,
                    out_hbm.at[pl.ds(base + sid * SLICE, SLICE)])


@jax.jit
def _fm_call(xt, bias128, tabt, tail):
    mesh = plsc.VectorSubcoreMesh(core_axis_name="c", subcore_axis_name="s",
                                  num_cores=NC, num_subcores=NS)
    fn = pl.kernel(
        _fm_body,
        out_type=jax.ShapeDtypeStruct((B,), jnp.float32),
        mesh=mesh,
        compiler_params=pltpu.CompilerParams(needs_layout_passes=False,
                                             use_tc_tiling_on_sc=True),
        scratch_types=[
            pltpu.VMEM((1, SLAB), jnp.float32),
            pltpu.VMEM((1, HALF), jnp.int32),
            pltpu.VMEM((1, HALF), jnp.float32),
            pltpu.VMEM((1, HALF), jnp.float32),
            pltpu.VMEM((SLICE,), jnp.float32),
            pltpu.VMEM((128,), jnp.float32),
            pltpu.VMEM((1, TAIL * D), jnp.float32),
            pltpu.VMEM_SHARED((NS // 2, HALF), jnp.float32),
            pltpu.SemaphoreType.DMA,
            pltpu.SemaphoreType.DMA,
        ],
    )
    return fn(xt, bias128, tabt, tail)


def kernel(x, additional, column, emb_table, bias):
    del additional, column
    xt = x.T
    tabt = emb_table.T
    bias128 = jnp.broadcast_to(bias.astype(jnp.float32), (128,))
    tail = emb_table[TAIL_START:, :].T.reshape(1, TAIL * D)
    return _fm_call(xt, bias128, tabt, tail)


# speedup vs baseline: 1.1270x; 1.0435x over previous
import numpy as np
import jax
import jax.numpy as jnp
from jax import lax
from jax.experimental import pallas as pl
from jax.experimental.pallas import tpu as pltpu, tpu_sc as plsc

B = 16384
F = 22
D = 16
W = 100000
NC, NS, L = 2, 16, 16
HALF = B // NC
SLICE = HALF // NS
NROWS = 2200000
_STARTS = [(f * W) // 128 * 128 for f in range(F)]
_SHIFT = [f * W - _STARTS[f] for f in range(F)]
SLAB = ((W + 127) // 128 + 1) * 128
_LEN = [SLAB] * (F - 1) + [99968]
TAIL = 64
TAIL_START = NROWS - TAIL
_COLS = [0, 1, 4, 5, 12, 17, 18, 19, 20] + list(range(26, 39))


def _fm_body(xt_hbm, bias_hbm, tab_hbm, tail_hbm, out_hbm,
             slab_v, idx_v, s_v, q_v, out_v, bias_v, tail_v, shared, sems,
             semi):
    cid = lax.axis_index("c")
    sid = lax.axis_index("s")
    base = cid * HALF

    pltpu.sync_copy(bias_hbm, bias_v)
    pltpu.sync_copy(tail_hbm, tail_v)
    bias_vec = bias_v[pl.ds(0, L)]
    zero_i = jnp.zeros((L,), jnp.int32)

    QT = 1
    QL = HALF

    NSTRIP = 4
    for f in range(F):
        ln = _LEN[f]
        sl_cuts = [ln * k // NSTRIP // 128 * 128 for k in range(NSTRIP)] + [ln]
        cps = []
        for k in range(NSTRIP):
            a, b = sl_cuts[k], sl_cuts[k + 1]
            cps.append(pltpu.async_copy(
                tab_hbm.at[pl.ds(sid, 1), pl.ds(_STARTS[f] + a, b - a)],
                slab_v.at[:, pl.ds(a, b - a)], sems))
        shift = _SHIFT[f]
        lim = _LEN[f]
        for qt in range(QT):
            cpi = pltpu.async_copy(
                xt_hbm.at[pl.ds(_COLS[f], 1),
                          pl.ds(base + qt * QL, QL)], idx_v, semi)
            if qt == 0:
                for cp in cps:
                    cp.wait()
            cpi.wait()
            qbase = qt * QL

            if f == 0:
                def init_body(g, _):
                    idx = idx_v[0, pl.ds(g * L, L)] + shift
                    val = plsc.load_gather(slab_v, [zero_i, idx])
                    sl = pl.ds(qbase + g * L, L)
                    s_v[0, sl] = val
                    q_v[0, sl] = val * val
                    return 0
                lax.fori_loop(0, QL // L, init_body, 0)
            elif f < F - 1:
                def acc_body(g2, _):
                    for u in range(4):
                        g = g2 * 4 + u
                        idx = idx_v[0, pl.ds(g * L, L)] + shift
                        val = plsc.load_gather(slab_v, [zero_i, idx])
                        sl = pl.ds(qbase + g * L, L)
                        plsc.addupdate(s_v.at[0, sl], val)
                        plsc.addupdate(q_v.at[0, sl], val * val)
                    return 0
                lax.fori_loop(0, QL // L // 4, acc_body, 0)
            else:
                def tail_body(g, _):
                    idx = idx_v[0, pl.ds(g * L, L)] + shift
                    in_slab = idx < lim
                    val_a = plsc.load_gather(
                        slab_v, [zero_i, jnp.minimum(idx, lim - 1)])
                    trow = jnp.clip(idx - lim, 0, TAIL - 1)
                    val_b = plsc.load_gather(tail_v,
                                             [zero_i, sid * TAIL + trow])
                    val = jnp.where(in_slab, val_a, val_b)
                    sl = pl.ds(qbase + g * L, L)
                    plsc.addupdate(s_v.at[0, sl], val)
                    plsc.addupdate(q_v.at[0, sl], val * val)
                    return 0
                lax.fori_loop(0, QL // L, tail_body, 0)

    def t_body(g2, _):
        for u in range(4):
            sl = pl.ds((g2 * 4 + u) * L, L)
            s = s_v[0, sl]
            s_v[0, sl] = s + 0.5 * (s * s - q_v[0, sl])
        return 0
    lax.fori_loop(0, HALF // L // 4, t_body, 0)

    def add_from_q(g2, _):
        for u in range(4):
            sl = pl.ds((g2 * 4 + u) * L, L)
            plsc.addupdate(s_v.at[0, sl], q_v[0, sl])
        return 0

    for lo in (8, 4, 2, 1):
        @pl.when((sid >= lo) & (sid < 2 * lo))
        def _write():
            pltpu.sync_copy(s_v, shared.at[pl.ds(sid - lo, 1)])
        plsc.subcore_barrier()

        @pl.when(sid < lo)
        def _combine():
            pltpu.sync_copy(shared.at[pl.ds(sid, 1)], q_v)
            lax.fori_loop(0, HALF // L // 4, add_from_q, 0)
        plsc.subcore_barrier()

    @pl.when(sid == 0)
    def _publish():
        pltpu.sync_copy(s_v, shared.at[pl.ds(0, 1)])
    plsc.subcore_barrier()
    pltpu.sync_copy(shared.at[pl.ds(0, 1), pl.ds(sid * SLICE, SLICE)],
                    slab_v.at[:, pl.ds(0, SLICE)])

    def out_body(g, _):
        z = slab_v[0, pl.ds(g * L, L)]
        y = 1.0 / (1.0 + jnp.exp(-(z + bias_vec)))
        out_v[pl.ds(g * L, L)] = y
        return 0
    lax.fori_loop(0, SLICE // L, out_body, 0)

    pltpu.sync_copy(out_v,
                    out_hbm.at[pl.ds(base + sid * SLICE, SLICE)])


@jax.jit
def _fm_call(xt, bias128, tabt, tail):
    mesh = plsc.VectorSubcoreMesh(core_axis_name="c", subcore_axis_name="s",
                                  num_cores=NC, num_subcores=NS)
    fn = pl.kernel(
        _fm_body,
        out_type=jax.ShapeDtypeStruct((B,), jnp.float32),
        mesh=mesh,
        compiler_params=pltpu.CompilerParams(needs_layout_passes=False,
                                             use_tc_tiling_on_sc=True),
        scratch_types=[
            pltpu.VMEM((1, SLAB), jnp.float32),
            pltpu.VMEM((1, HALF), jnp.int32),
            pltpu.VMEM((1, HALF), jnp.float32),
            pltpu.VMEM((1, HALF), jnp.float32),
            pltpu.VMEM((SLICE,), jnp.float32),
            pltpu.VMEM((128,), jnp.float32),
            pltpu.VMEM((1, TAIL * D), jnp.float32),
            pltpu.VMEM_SHARED((NS // 2, HALF), jnp.float32),
            pltpu.SemaphoreType.DMA,
            pltpu.SemaphoreType.DMA,
        ],
    )
    return fn(xt, bias128, tabt, tail)


def kernel(x, additional, column, emb_table, bias):
    del additional, column
    xt = x.T
    tabt = emb_table.T
    bias128 = jnp.broadcast_to(bias.astype(jnp.float32), (128,))
    tail = emb_table[TAIL_START:, :].T.reshape(1, TAIL * D)
    return _fm_call(xt, bias128, tabt, tail)
